# initial kernel scaffold (unmeasured)
import jax
import jax.numpy as jnp
from jax import lax
from jax.experimental import pallas as pl
from jax.experimental.pallas import tpu as pltpu

N_DEV = 32


def kernel(x, w_mat):
    m_glob, k_per = x.shape
    k_glob, n = w_mat.shape
    m_per = m_glob // N_DEV
    assert k_glob // N_DEV == k_per

    def body(x_ref, w_ref, out_ref, comm_ref, amax_ref,
             send_sems, recv_sems, asend_sems, arecv_sems):
        my_i = lax.axis_index("i")

        descs = []
        for off in range(1, N_DEV):
            dst = (my_i + off) % N_DEV
            d = pltpu.make_async_remote_copy(
                src_ref=x_ref.at[pl.ds(dst * m_per, m_per), :],
                dst_ref=comm_ref.at[off],
                send_sem=send_sems.at[off],
                recv_sem=recv_sems.at[off],
                device_id=(dst,),
                device_id_type=pl.DeviceIdType.MESH,
            )
            d.start()
            descs.append(d)

        comm_ref[0] = x_ref[pl.ds(my_i * m_per, m_per), :]

        for d in descs:
            d.wait()

        acc = jnp.zeros((m_per, n), jnp.float32)
        for off in range(N_DEV):
            src = (my_i - off) % N_DEV
            wblk = w_ref[pl.ds(src * k_per, k_per), :]
            acc += jnp.dot(comm_ref[off], wblk,
                           preferred_element_type=jnp.float32)

        y = jnp.maximum(acc, 0.0)

        amax_ref[0] = jnp.full((1, 128), jnp.max(y), jnp.float32)
        adescs = []
        for off in range(1, N_DEV):
            dst = (my_i + off) % N_DEV
            d = pltpu.make_async_remote_copy(
                src_ref=amax_ref.at[0],
                dst_ref=amax_ref.at[off],
                send_sem=asend_sems.at[off],
                recv_sem=arecv_sems.at[off],
                device_id=(dst,),
                device_id_type=pl.DeviceIdType.MESH,
            )
            d.start()
            adescs.append(d)
        for d in adescs:
            d.wait()

        gmax = jnp.max(amax_ref[...])
        scale = gmax / 127.0
        q = jnp.clip(jnp.round(y / scale), -127.0, 127.0)
        out_ref[...] = q * scale

    return pl.pallas_call(
        body,
        out_shape=jax.ShapeDtypeStruct((m_per, n), jnp.float32),
        in_specs=[
            pl.BlockSpec(memory_space=pltpu.VMEM),
            pl.BlockSpec(memory_space=pltpu.VMEM),
        ],
        out_specs=pl.BlockSpec(memory_space=pltpu.VMEM),
        scratch_shapes=[
            pltpu.VMEM((N_DEV, m_per, k_per), x.dtype),
            pltpu.VMEM((N_DEV, 1, 128), jnp.float32),
            pltpu.SemaphoreType.DMA((N_DEV,)),
            pltpu.SemaphoreType.DMA((N_DEV,)),
            pltpu.SemaphoreType.DMA((N_DEV,)),
            pltpu.SemaphoreType.DMA((N_DEV,)),
        ],
        compiler_params=pltpu.CompilerParams(collective_id=0),
    )(x, w_mat)


# baseline (device time: 65880 ns/iter reference)
import jax
import jax.numpy as jnp
from jax import lax
from jax.experimental import pallas as pl
from jax.experimental.pallas import tpu as pltpu

N_DEV = 32


def kernel(x, w_mat):
    m_glob, k_per = x.shape
    k_glob, n = w_mat.shape
    m_per = m_glob // N_DEV
    assert k_glob // N_DEV == k_per

    def body(x_ref, w_ref, out_ref, comm_ref, amax_ref,
             send_sems, recv_sems, asend_sems, arecv_sems):
        my_i = lax.axis_index("i")

        descs = []
        for off in range(1, N_DEV):
            dst = (my_i + off) % N_DEV
            d = pltpu.make_async_remote_copy(
                src_ref=x_ref.at[pl.ds(dst * m_per, m_per), :],
                dst_ref=comm_ref.at[off],
                send_sem=send_sems.at[off],
                recv_sem=recv_sems.at[off],
                device_id=(dst,),
                device_id_type=pl.DeviceIdType.MESH,
            )
            d.start()
            descs.append(d)

        comm_ref[0] = x_ref[pl.ds(my_i * m_per, m_per), :]

        for d in descs:
            d.wait()

        acc = jnp.zeros((m_per, n), jnp.float32)
        for off in range(N_DEV):
            src = (my_i - off) % N_DEV
            wblk = w_ref[pl.ds(src * k_per, k_per), :]
            acc += jnp.dot(comm_ref[off], wblk,
                           preferred_element_type=jnp.float32)

        y = jnp.maximum(acc, 0.0)

        amax_ref[0] = jnp.full((1, 128), jnp.max(y), jnp.float32)
        adescs = []
        for off in range(1, N_DEV):
            dst = (my_i + off) % N_DEV
            d = pltpu.make_async_remote_copy(
                src_ref=amax_ref.at[0],
                dst_ref=amax_ref.at[off],
                send_sem=asend_sems.at[off],
                recv_sem=arecv_sems.at[off],
                device_id=(dst,),
                device_id_type=pl.DeviceIdType.MESH,
            )
            d.start()
            adescs.append(d)
        for d in adescs:
            d.wait()

        gmax = jnp.max(amax_ref[...])
        scale = gmax / 127.0
        q = jnp.clip(jnp.round(y / scale), -127.0, 127.0)
        out_ref[...] = q * scale

    return pl.pallas_call(
        body,
        out_shape=jax.ShapeDtypeStruct((m_per, n), jnp.float32),
        in_specs=[
            pl.BlockSpec(memory_space=pltpu.VMEM),
            pl.BlockSpec(memory_space=pltpu.VMEM),
        ],
        out_specs=pl.BlockSpec(memory_space=pltpu.VMEM),
        scratch_shapes=[
            pltpu.VMEM((N_DEV, m_per, k_per), x.dtype),
            pltpu.VMEM((N_DEV, 1, 128), jnp.float32),
            pltpu.SemaphoreType.DMA((N_DEV,)),
            pltpu.SemaphoreType.DMA((N_DEV,)),
            pltpu.SemaphoreType.DMA((N_DEV,)),
            pltpu.SemaphoreType.DMA((N_DEV,)),
        ],
        compiler_params=pltpu.CompilerParams(
            vmem_limit_bytes=96 * 1024 * 1024,
        ),
    )(x, w_mat)


# device time: 46932 ns/iter; 1.4037x vs baseline; 1.4037x over previous
import jax
import jax.numpy as jnp
from jax import lax
from jax.experimental import pallas as pl
from jax.experimental.pallas import tpu as pltpu

N_DEV = 32
KC = 512
NC = 4096 // KC


def kernel(x, w_mat):
    m_glob, k_per = x.shape
    k_glob, n = w_mat.shape
    m_per = m_glob // N_DEV
    assert k_glob // N_DEV == k_per

    def body(x_ref, w_hbm, out_ref, xbf_ref, xg_ref, wbuf_ref, amax_ref,
             send_sems, recv_sems, asend_sems, arecv_sems, wdma_sems):
        my_i = lax.axis_index("i")

        xbf_ref[...] = x_ref[...].astype(jnp.bfloat16)

        def wdma(c, slot):
            return pltpu.make_async_copy(
                w_hbm.at[pl.ds(c * KC, KC), :],
                wbuf_ref.at[slot],
                wdma_sems.at[slot],
            )

        wdma(0, 0).start()

        descs = []
        for off in range(1, N_DEV):
            dst = (my_i + off) % N_DEV
            d = pltpu.make_async_remote_copy(
                src_ref=xbf_ref.at[pl.ds(dst * m_per, m_per), :],
                dst_ref=xg_ref.at[:, pl.ds(my_i * k_per, k_per)],
                send_sem=send_sems.at[off],
                recv_sem=recv_sems.at[off],
                device_id=(dst,),
                device_id_type=pl.DeviceIdType.MESH,
            )
            d.start()
            descs.append(d)

        xg_ref[:, pl.ds(my_i * k_per, k_per)] = (
            xbf_ref[pl.ds(my_i * m_per, m_per), :])

        wdma(1, 1).start()

        for d in descs:
            d.wait()

        acc = jnp.zeros((m_per, n), jnp.float32)
        for c in range(NC):
            wdma(c, c % 2).wait()
            wchunk = wbuf_ref[c % 2]
            if c + 2 < NC:
                wdma(c + 2, c % 2).start()
            xchunk = xg_ref[:, pl.ds(c * KC, KC)].astype(jnp.float32)
            acc += jnp.dot(xchunk, wchunk,
                           preferred_element_type=jnp.float32)

        y = jnp.maximum(acc, 0.0)

        amax_ref[0] = jnp.full((1, 128), jnp.max(y), jnp.float32)
        adescs = []
        for off in range(1, N_DEV):
            dst = (my_i + off) % N_DEV
            d = pltpu.make_async_remote_copy(
                src_ref=amax_ref.at[0],
                dst_ref=amax_ref.at[off],
                send_sem=asend_sems.at[off],
                recv_sem=arecv_sems.at[off],
                device_id=(dst,),
                device_id_type=pl.DeviceIdType.MESH,
            )
            d.start()
            adescs.append(d)
        for d in adescs:
            d.wait()

        gmax = jnp.max(amax_ref[...])
        scale = gmax / 127.0
        q = jnp.clip(jnp.round(y / scale), -127.0, 127.0)
        out_ref[...] = q * scale

    return pl.pallas_call(
        body,
        out_shape=jax.ShapeDtypeStruct((m_per, n), jnp.float32),
        in_specs=[
            pl.BlockSpec(memory_space=pltpu.VMEM),
            pl.BlockSpec(memory_space=pltpu.MemorySpace.HBM),
        ],
        out_specs=pl.BlockSpec(memory_space=pltpu.VMEM),
        scratch_shapes=[
            pltpu.VMEM((m_glob, k_per), jnp.bfloat16),
            pltpu.VMEM((m_per, k_glob), jnp.bfloat16),
            pltpu.VMEM((2, KC, n), jnp.float32),
            pltpu.VMEM((N_DEV, 1, 128), jnp.float32),
            pltpu.SemaphoreType.DMA((N_DEV,)),
            pltpu.SemaphoreType.DMA((N_DEV,)),
            pltpu.SemaphoreType.DMA((N_DEV,)),
            pltpu.SemaphoreType.DMA((N_DEV,)),
            pltpu.SemaphoreType.DMA((2,)),
        ],
        compiler_params=pltpu.CompilerParams(
            vmem_limit_bytes=96 * 1024 * 1024,
        ),
    )(x, w_mat)


# device time: 41981 ns/iter; 1.5693x vs baseline; 1.1179x over previous
import jax
import jax.numpy as jnp
from jax import lax
from jax.experimental import pallas as pl
from jax.experimental.pallas import tpu as pltpu

N_DEV = 32
W_SLOTS = 4


def kernel(x, w_mat):
    m_glob, k_per = x.shape
    k_glob, n = w_mat.shape
    m_per = m_glob // N_DEV
    assert k_glob // N_DEV == k_per

    def body(x_ref, w_hbm, out_ref, xbf_ref, xg_ref, wbuf_ref, amax_ref,
             send_sems, recv_sems, asend_sems, arecv_sems, wdma_sems):
        my_i = lax.axis_index("i")

        xbf_ref[...] = x_ref[...].astype(jnp.bfloat16)

        def wdma(off, slot):
            src = (my_i - off) % N_DEV
            return pltpu.make_async_copy(
                w_hbm.at[pl.ds(src * k_per, k_per), :],
                wbuf_ref.at[slot],
                wdma_sems.at[slot],
            )

        for c in range(W_SLOTS):
            wdma(c, c).start()

        descs = []
        for off in range(1, N_DEV):
            dst = (my_i + off) % N_DEV
            d = pltpu.make_async_remote_copy(
                src_ref=xbf_ref.at[pl.ds(dst * m_per, m_per), :],
                dst_ref=xg_ref.at[:, pl.ds(my_i * k_per, k_per)],
                send_sem=send_sems.at[off],
                recv_sem=recv_sems.at[off],
                device_id=(dst,),
                device_id_type=pl.DeviceIdType.MESH,
            )
            d.start()
            descs.append(d)

        xg_ref[:, pl.ds(my_i * k_per, k_per)] = (
            xbf_ref[pl.ds(my_i * m_per, m_per), :])

        acc = jnp.zeros((m_per, n), jnp.float32)
        for off in range(N_DEV):
            slot = off % W_SLOTS
            wdma(off, slot).wait()
            if off > 0:
                descs[off - 1].wait()
            src = (my_i - off) % N_DEV
            xchunk = xg_ref[:, pl.ds(src * k_per, k_per)].astype(jnp.float32)
            acc += jnp.dot(xchunk, wbuf_ref[slot],
                           preferred_element_type=jnp.float32)
            if off + W_SLOTS < N_DEV:
                wdma(off + W_SLOTS, slot).start()

        y = jnp.maximum(acc, 0.0)

        amax_ref[0] = jnp.full((1, 128), jnp.max(y), jnp.float32)
        adescs = []
        for off in range(1, N_DEV):
            dst = (my_i + off) % N_DEV
            d = pltpu.make_async_remote_copy(
                src_ref=amax_ref.at[0],
                dst_ref=amax_ref.at[off],
                send_sem=asend_sems.at[off],
                recv_sem=arecv_sems.at[off],
                device_id=(dst,),
                device_id_type=pl.DeviceIdType.MESH,
            )
            d.start()
            adescs.append(d)
        for d in adescs:
            d.wait()

        gmax = jnp.max(amax_ref[...])
        scale = gmax / 127.0
        q = jnp.clip(jnp.round(y / scale), -127.0, 127.0)
        out_ref[...] = q * scale

    return pl.pallas_call(
        body,
        out_shape=jax.ShapeDtypeStruct((m_per, n), jnp.float32),
        in_specs=[
            pl.BlockSpec(memory_space=pltpu.VMEM),
            pl.BlockSpec(memory_space=pltpu.MemorySpace.HBM),
        ],
        out_specs=pl.BlockSpec(memory_space=pltpu.VMEM),
        scratch_shapes=[
            pltpu.VMEM((m_glob, k_per), jnp.bfloat16),
            pltpu.VMEM((m_per, k_glob), jnp.bfloat16),
            pltpu.VMEM((W_SLOTS, k_per, n), jnp.float32),
            pltpu.VMEM((N_DEV, 1, 128), jnp.float32),
            pltpu.SemaphoreType.DMA((N_DEV,)),
            pltpu.SemaphoreType.DMA((N_DEV,)),
            pltpu.SemaphoreType.DMA((N_DEV,)),
            pltpu.SemaphoreType.DMA((N_DEV,)),
            pltpu.SemaphoreType.DMA((2,)),
        ],
        compiler_params=pltpu.CompilerParams(
            vmem_limit_bytes=96 * 1024 * 1024,
        ),
    )(x, w_mat)
